# baseline (speedup 1.0000x reference)
import jax
import jax.numpy as jnp
from jax.experimental import pallas as pl

_EGO = 0.01831563888873418
_EGE = 0.36787944117144233
_NEG = -1e30
_B = 16
_HB = 8
_L = 256
_CHUNK = 32
_NCHUNK = 16
_NH = 2


def _shiftn_by(x, k):
    return jnp.concatenate(
        [jnp.full((x.shape[0], k), _NEG, x.dtype), x[:, :-k]], axis=1)


def _shift0(x):
    return jnp.concatenate(
        [jnp.zeros((x.shape[0], 1), x.dtype), x[:, :-1]], axis=1)


def _rotk(x, k):
    k = k % _L
    if k == 0:
        return x
    return jnp.concatenate([x[:, -k:], x[:, :-k]], axis=1)


def _sel4(t, v):
    return jnp.where(t == 0, v[0],
           jnp.where(t == 1, v[1],
           jnp.where(t == 2, v[2], v[3])))


def _sw_kernel(predT_ref, v0_ref, out_ref):
    predT = predT_ref[...]
    vfull = v0_ref[...]
    lane = jax.lax.broadcasted_iota(jnp.int32, (_HB, _L), 1)
    vmask = lane < (_L - 1)
    zero = jnp.zeros((_HB, _L), jnp.float32)

    p0p = []
    ep1 = []
    vh = []
    for h in range(_NH):
        sl = slice(h * _HB, (h + 1) * _HB)
        p0p.append([jnp.where(vmask, jnp.maximum(predT[p][sl], 0.0), zero)
                    for p in range(4)])
        ep1.append([jnp.where(vmask,
                              jnp.exp(jnp.concatenate(
                                  [predT[p][sl][:, 1:], predT[p][sl][:, :1]],
                                  axis=1)),
                              zero) for p in range(4)])
        vh.append(vfull[sl])

    def chunk(i, carry):
        st = [list(carry[h * 9:(h + 1) * 9]) for h in range(_NH)]
        vcs = list(carry[_NH * 9:])
        dsh = [None] * _NH
        e00 = [None] * _NH
        for h in range(_NH):
            mx, ea1, er1, ssq1, seg1, seg2, eg1, w1, acc = st[h]
            mxe = mx + jnp.log(jnp.maximum(eg1, 1.0))
            mxn = mxe
            for j in (1, 2, 4, 8, 16, 32):
                mxn = jnp.maximum(mxn, _shiftn_by(mxn, j))
            rb = jnp.exp(mx - mxn)
            dsh[h] = jnp.exp(jnp.minimum(_shiftn_by(mxn, 1) - mxn, 80.0))
            e00[h] = jnp.exp(-mxn)
            st[h] = [mxn, ea1 * rb, er1 * rb, ssq1 * rb, seg1 * rb,
                     seg2 * rb, eg1 * rb, w1 * rb, acc * rb]

        d_base = i * _CHUNK
        ld0 = lane - d_base
        ws = [[_rotk(vcs[h], k) for k in range(_CHUNK + 1)]
              for h in range(_NH)]
        for k in range(_CHUNK):
            ld = ld0 - k
            mask = (ld <= 0) & (ld >= -254)
            for h in range(_NH):
                mxn, ea1, er1, ssq1, seg1, seg2, eg1, w1, acc = st[h]
                sp = jnp.where(mask, _sel4(ws[h][k], p0p[h]), zero)
                esmx = jnp.where(mask, jnp.exp(sp), zero)
                exe = jnp.where(mask, _sel4(ws[h][k + 1], ep1[h]), zero)
                ea0 = esmx * (seg2 + e00[h])
                er0 = w1 + _EGE * er1
                ed0 = ssq1
                eg0 = ea0 + er0 + ed0
                w0 = _EGO * ea0
                sq0 = w0 + _EGO * er0 + _EGE * ed0
                acc = acc + eg0 * exe
                st[h] = [mxn, ea0, er0, _shift0(sq0) * dsh[h],
                         _shift0(eg0) * dsh[h], seg1, eg0, w0, acc]
        for h in range(_NH):
            vcs[h] = ws[h][_CHUNK]
        flat = []
        for h in range(_NH):
            flat.extend(st[h])
        return tuple(flat) + tuple(vcs)

    zi = [predT[0][h * _HB:(h + 1) * _HB] * 0.0 for h in range(_NH)]
    init = []
    for h in range(_NH):
        init.extend([zi[h]] * 9)
    init = tuple(init) + tuple(vh)
    out = jax.lax.fori_loop(0, _NCHUNK, chunk, init)

    fins = []
    for h in range(_NH):
        mx = out[h * 9]
        acc = out[h * 9 + 8]
        t = mx + jnp.log(jnp.maximum(acc, 1e-35))
        mb = jnp.max(t, axis=1, keepdims=True)
        sb = jnp.sum(jnp.exp(t - mb), axis=1, keepdims=True)
        fins.append(jnp.sum(mb + jnp.log(sb)))
    out_ref[...] = jnp.full((1, 1), -(fins[0] + fins[1]) * (1.0 / _B),
                            jnp.float32)


def _prep(predictions, targets):
    predT = jnp.transpose(predictions.astype(jnp.float32), (2, 0, 1))
    t = targets.astype(jnp.int32)
    v0 = jnp.concatenate([t[:, :1], jnp.flip(t[:, 1:], axis=1)], axis=1)
    return predT, v0


@jax.jit
def kernel(predictions, targets):
    predT, v0 = _prep(predictions, targets)
    out = pl.pallas_call(
        _sw_kernel,
        out_shape=jax.ShapeDtypeStruct((1, 1), jnp.float32),
    )(predT, v0)
    return out[0, 0]


# speedup vs baseline: 1.6682x; 1.6682x over previous
import jax
import jax.numpy as jnp
from jax.experimental import pallas as pl

_EGO = 0.01831563888873418
_EGE = 0.36787944117144233
_B = 16
_HB = 8
_L = 256
_CHUNK = 32
_NCHUNK = 16
_NH = 2


def _sh(x, k):
    return jnp.concatenate(
        [jnp.zeros((x.shape[0], k), x.dtype), x[:, :-k]], axis=1)


def _rotk(x, k):
    k = k % _L
    if k == 0:
        return x
    return jnp.concatenate([x[:, -k:], x[:, :-k]], axis=1)


def _sel4(t, v):
    return jnp.where(t == 0, v[0],
           jnp.where(t == 1, v[1],
           jnp.where(t == 2, v[2], v[3])))


def _sw_kernel(predT_ref, v0_ref, out_ref):
    predT = predT_ref[...]
    vfull = v0_ref[...]
    lane = jax.lax.broadcasted_iota(jnp.int32, (_HB, _L), 1)
    vmask = lane < (_L - 1)
    zero = jnp.zeros((_HB, _L), jnp.float32)

    ep0 = []
    ep0s = []
    ep1 = []
    vh = []
    for h in range(_NH):
        sl = slice(h * _HB, (h + 1) * _HB)
        t0 = [jnp.where(vmask, jnp.exp(predT[p][sl]), zero) for p in range(4)]
        ep0.append(t0)
        ep0s.append([_sh(x, 1) for x in t0])
        ep1.append([jnp.where(vmask,
                              jnp.exp(jnp.concatenate(
                                  [predT[p][sl][:, 1:], predT[p][sl][:, :1]],
                                  axis=1)),
                              zero) for p in range(4)])
        vh.append(vfull[sl])

    def chunk(i, carry):
        st = [list(carry[h * 8:(h + 1) * 8]) for h in range(_NH)]
        vcs = []
        d_base = i * _CHUNK
        ld0 = lane - d_base

        e00 = [None] * _NH
        for h in range(_NH):
            mx, ea1, er1, sq1, eg1, eg2, acc, vc = st[h]
            m = jnp.maximum(jnp.max(eg1, axis=1, keepdims=True), 1.0)
            rr = 1.0 / m
            mx = mx + jnp.log(m)
            e00[h] = jnp.exp(-mx)
            st[h] = [mx, ea1 * rr, er1 * rr, sq1 * rr, eg1 * rr,
                     eg2 * rr, acc * rr, vc]

        ws = [[_rotk(st[h][7], k) for k in range(_CHUNK + 1)]
              for h in range(_NH)]
        masks = [((ld0 - k) <= 0) & ((ld0 - k) >= -254)
                 for k in range(_CHUNK)]
        esmx = [[jnp.where(masks[k], _sel4(ws[h][k], ep0[h]), zero)
                 for k in range(_CHUNK)] for h in range(_NH)]
        exe = [[jnp.where(masks[k], _sel4(ws[h][k + 1], ep1[h]), zero)
                for k in range(_CHUNK)] for h in range(_NH)]
        sesmx = [[jnp.where(masks[k + 1], _sel4(ws[h][k + 1], ep0s[h]), zero)
                  if k % 2 == 0 else None
                  for k in range(_CHUNK)] for h in range(_NH)]

        for k in range(0, _CHUNK, 2):
            for h in range(_NH):
                mx, ea1, er1, sq1, eg1, eg2, acc, vc = st[h]
                v1 = _EGO * ea1 + _EGE * er1
                a1 = _sh(eg2, 1)
                b1 = _sh(eg1, 1)
                c1 = _sh(sq1, 1)
                d2 = _sh(sq1, 2)
                a2 = _sh(eg2, 2)
                vv = _sh(v1, 1)
                ea_d = esmx[h][k] * (a1 + e00[h])
                er_d = v1
                s_d = ea_d + er_d
                eg_d = s_d + c1
                sq_d = _EGO * s_d + _EGE * c1
                acc = acc + eg_d * exe[h][k]
                sea = sesmx[h][k] * (a2 + e00[h])
                ea_o = esmx[h][k + 1] * (b1 + e00[h])
                er_o = _EGO * ea_d + _EGE * er_d
                ed_o = _EGO * (sea + vv) + _EGE * d2
                s_o = ea_o + er_o
                eg_o = s_o + ed_o
                sq_o = _EGO * s_o + _EGE * ed_o
                acc = acc + eg_o * exe[h][k + 1]
                st[h] = [mx, ea_o, er_o, sq_o, eg_o, eg_d, acc, vc]
        for h in range(_NH):
            st[h][7] = ws[h][_CHUNK]
        flat = []
        for h in range(_NH):
            flat.extend(st[h])
        return tuple(flat)

    zi = [predT[0][h * _HB:(h + 1) * _HB] * 0.0 for h in range(_NH)]
    zc = [z[:, :1] for z in zi]
    init = []
    for h in range(_NH):
        init.extend([zc[h], zi[h], zi[h], zi[h], zi[h], zi[h], zi[h], vh[h]])
    init = tuple(init)
    out = jax.lax.fori_loop(0, _NCHUNK, chunk, init)

    fins = []
    for h in range(_NH):
        mx = out[h * 8]
        acc = out[h * 8 + 6]
        t = jnp.log(jnp.maximum(acc, 1e-35))
        mb = jnp.max(t, axis=1, keepdims=True)
        sb = jnp.sum(jnp.exp(t - mb), axis=1, keepdims=True)
        fins.append(jnp.sum(mx + mb + jnp.log(sb)))
    out_ref[...] = jnp.full((1, 1), -(fins[0] + fins[1]) * (1.0 / _B),
                            jnp.float32)


def _prep(predictions, targets):
    predT = jnp.transpose(predictions.astype(jnp.float32), (2, 0, 1))
    t = targets.astype(jnp.int32)
    v0 = jnp.concatenate([t[:, :1], jnp.flip(t[:, 1:], axis=1)], axis=1)
    return predT, v0


@jax.jit
def kernel(predictions, targets):
    predT, v0 = _prep(predictions, targets)
    out = pl.pallas_call(
        _sw_kernel,
        out_shape=jax.ShapeDtypeStruct((1, 1), jnp.float32),
    )(predT, v0)
    return out[0, 0]


# speedup vs baseline: 1.9993x; 1.1984x over previous
import jax
import jax.numpy as jnp
from jax.experimental import pallas as pl

_EGO = 0.01831563888873418
_EGE = 0.36787944117144233
_B = 16
_HB = 8
_L = 256
_CHUNK = 32
_NCHUNK = 16
_NH = 2


def _sh(x, k):
    return jnp.concatenate(
        [jnp.zeros((x.shape[0], k), x.dtype), x[:, :-k]], axis=1)


def _rotk(x, k):
    k = k % _L
    if k == 0:
        return x
    return jnp.concatenate([x[:, -k:], x[:, :-k]], axis=1)


def _sel4(t, v):
    return jnp.where(t == 0, v[0],
           jnp.where(t == 1, v[1],
           jnp.where(t == 2, v[2], v[3])))


def _sw_kernel(predT_ref, v0_ref, out_ref):
    predT = predT_ref[...]
    vfull = v0_ref[...]
    lane = jax.lax.broadcasted_iota(jnp.int32, (_HB, _L), 1)
    vmask = lane < (_L - 1)
    zero = jnp.zeros((_HB, _L), jnp.float32)

    ep0 = []
    ep0s = []
    ep1 = []
    vh = []
    for h in range(_NH):
        sl = slice(h * _HB, (h + 1) * _HB)
        t0 = [jnp.where(vmask, jnp.exp(predT[p][sl]), zero) for p in range(4)]
        ep0.append(t0)
        ep0s.append([[_sh(x, j) for x in t0] for j in (1, 2, 3)])
        ep1.append([jnp.where(vmask,
                              jnp.exp(jnp.concatenate(
                                  [predT[p][sl][:, 1:], predT[p][sl][:, :1]],
                                  axis=1)),
                              zero) for p in range(4)])
        vh.append(vfull[sl])

    def chunk(i, carry):
        st = [list(carry[h * 8:(h + 1) * 8]) for h in range(_NH)]
        d_base = i * _CHUNK
        ld0 = lane - d_base

        e00 = [None] * _NH
        for h in range(_NH):
            mx, ea1, er1, sq1, eg1, eg2, acc, vc = st[h]
            m = jnp.maximum(jnp.max(eg1, axis=1, keepdims=True), 1.0)
            rr = 1.0 / m
            mx = mx + jnp.log(m)
            e00[h] = jnp.exp(-mx)
            st[h] = [mx, ea1 * rr, er1 * rr, sq1 * rr, eg1 * rr,
                     eg2 * rr, acc * rr, vc]

        ws = [[_rotk(st[h][7], k) for k in range(_CHUNK + 1)]
              for h in range(_NH)]
        masks = [((ld0 - k) <= 0) & ((ld0 - k) >= -254)
                 for k in range(_CHUNK + 3)]
        esmx = [[jnp.where(masks[k], _sel4(ws[h][k], ep0[h]), zero)
                 for k in range(_CHUNK)] for h in range(_NH)]
        exe = [[jnp.where(masks[k], _sel4(ws[h][k + 1], ep1[h]), zero)
                for k in range(_CHUNK)] for h in range(_NH)]

        def sesel(h, k, j):
            w = ws[h][k + j] if k + j <= _CHUNK else _rotk(ws[h][_CHUNK],
                                                           k + j - _CHUNK)
            return jnp.where(masks[k + j], _sel4(w, ep0s[h][j - 1]), zero)

        for k in range(0, _CHUNK, 4):
            for h in range(_NH):
                mx, ea1, er1, sq1, eg1, eg2, acc, vc = st[h]
                e0h = e00[h]
                v1 = _EGO * ea1 + _EGE * er1
                q1 = _sh(sq1, 1)
                q2 = _sh(sq1, 2)
                q3 = _sh(sq1, 3)
                q4 = _sh(sq1, 4)
                g21 = _sh(eg2, 1)
                g22 = _sh(eg2, 2)
                g23 = _sh(eg2, 3)
                g24 = _sh(eg2, 4)
                g11 = _sh(eg1, 1)
                g12 = _sh(eg1, 2)
                g13 = _sh(eg1, 3)
                w1s = _sh(v1, 1)
                w2s = _sh(v1, 2)
                w3s = _sh(v1, 3)
                ea_d = esmx[h][k] * (g21 + e0h)
                er_d = v1
                s_d = ea_d + er_d
                eg_d = s_d + q1
                acc = acc + eg_d * exe[h][k]
                sea_d = sesel(h, k, 1) * (g22 + e0h)
                s2ea_d = sesel(h, k, 2) * (g23 + e0h)
                s3ea_d = sesel(h, k, 3) * (g24 + e0h)
                ss_d = sea_d + w1s
                s2s_d = s2ea_d + w2s
                s3s_d = s3ea_d + w3s
                seg_d = ss_d + q2
                s2eg_d = s2s_d + q3
                ssq_d = _EGO * ss_d + _EGE * q2
                s2sq_d = _EGO * s2s_d + _EGE * q3
                s3sq_d = _EGO * s3s_d + _EGE * q4
                ea_1 = esmx[h][k + 1] * (g11 + e0h)
                er_1 = _EGO * ea_d + _EGE * er_d
                s_1 = ea_1 + er_1
                eg_1 = s_1 + ssq_d
                acc = acc + eg_1 * exe[h][k + 1]
                sea_1 = sesel(h, k + 1, 1) * (g12 + e0h)
                s2ea_1 = sesel(h, k + 1, 2) * (g13 + e0h)
                ser_1 = _EGO * sea_d + _EGE * w1s
                s2er_1 = _EGO * s2ea_d + _EGE * w2s
                ss_1 = sea_1 + ser_1
                s2s_1 = s2ea_1 + s2er_1
                seg_1 = ss_1 + s2sq_d
                ssq_1 = _EGO * ss_1 + _EGE * s2sq_d
                s2sq_1 = _EGO * s2s_1 + _EGE * s3sq_d
                ea_2 = esmx[h][k + 2] * (seg_d + e0h)
                er_2 = _EGO * ea_1 + _EGE * er_1
                s_2 = ea_2 + er_2
                eg_2v = s_2 + ssq_1
                acc = acc + eg_2v * exe[h][k + 2]
                sea_2 = sesel(h, k + 2, 1) * (s2eg_d + e0h)
                ser_2 = _EGO * sea_1 + _EGE * ser_1
                ss_2 = sea_2 + ser_2
                ssq_2 = _EGO * ss_2 + _EGE * s2sq_1
                ea_3 = esmx[h][k + 3] * (seg_1 + e0h)
                er_3 = _EGO * ea_2 + _EGE * er_2
                s_3 = ea_3 + er_3
                eg_3 = s_3 + ssq_2
                sq_3 = _EGO * s_3 + _EGE * ssq_2
                acc = acc + eg_3 * exe[h][k + 3]
                st[h] = [mx, ea_3, er_3, sq_3, eg_3, eg_2v, acc, vc]
        for h in range(_NH):
            st[h][7] = ws[h][_CHUNK]
        flat = []
        for h in range(_NH):
            flat.extend(st[h])
        return tuple(flat)

    zi = [predT[0][h * _HB:(h + 1) * _HB] * 0.0 for h in range(_NH)]
    zc = [z[:, :1] for z in zi]
    init = []
    for h in range(_NH):
        init.extend([zc[h], zi[h], zi[h], zi[h], zi[h], zi[h], zi[h], vh[h]])
    out = jax.lax.fori_loop(0, _NCHUNK, chunk, tuple(init))

    fins = []
    for h in range(_NH):
        mx = out[h * 8]
        acc = out[h * 8 + 6]
        t = jnp.log(jnp.maximum(acc, 1e-35))
        mb = jnp.max(t, axis=1, keepdims=True)
        sb = jnp.sum(jnp.exp(t - mb), axis=1, keepdims=True)
        fins.append(jnp.sum(mx + mb + jnp.log(sb)))
    out_ref[...] = jnp.full((1, 1), -(fins[0] + fins[1]) * (1.0 / _B),
                            jnp.float32)


def _prep(predictions, targets):
    predT = jnp.transpose(predictions.astype(jnp.float32), (2, 0, 1))
    t = targets.astype(jnp.int32)
    v0 = jnp.concatenate([t[:, :1], jnp.flip(t[:, 1:], axis=1)], axis=1)
    return predT, v0


@jax.jit
def kernel(predictions, targets):
    predT, v0 = _prep(predictions, targets)
    out = pl.pallas_call(
        _sw_kernel,
        out_shape=jax.ShapeDtypeStruct((1, 1), jnp.float32),
    )(predT, v0)
    return out[0, 0]
